# baseline (speedup 1.0000x reference)
import jax
import jax.numpy as jnp
from jax.experimental import pallas as pl
from jax.experimental.pallas import tpu as pltpu

IN_ROWS = 25000
OUT_ROWS = 5000
SUBS = IN_ROWS // OUT_ROWS


def _fused_mlp_block(x_ref, w1_ref, b1_ref, gamma_ref, beta_ref, w2_ref,
                     b2_ref, out_ref):
    w2 = w2_ref[...]
    w2g = w2 * gamma_ref[...]
    b2f = b2_ref[...] + jax.lax.dot_general(beta_ref[...], w2,
                                            (((1,), (1,)), ((), ())),
                                            preferred_element_type=jnp.float32)
    j = pl.program_id(0) % SUBS
    xs = x_ref[pl.ds(j * OUT_ROWS, OUT_ROWS), :]
    h = jax.lax.dot_general(xs, w1_ref[...],
                            (((1,), (1,)), ((), ())),
                            preferred_element_type=jnp.float32)
    h = h + b1_ref[...]
    h = jnp.maximum(h, 0.01 * h)
    mu = jnp.mean(h, axis=-1, keepdims=True)
    d = h - mu
    var = jnp.mean(d * d, axis=-1, keepdims=True)
    hn = d * jax.lax.rsqrt(var + 1e-5)
    out = jax.lax.dot_general(hn, w2g,
                              (((1,), (1,)), ((), ())),
                              preferred_element_type=jnp.float32)
    out_ref[...] = out + b2f


@jax.jit
def kernel(x, W1, b1, gamma, beta, W2, b2):
    n, din = x.shape
    hid = W1.shape[0]
    dout = W2.shape[0]
    grid = (n // OUT_ROWS,)

    b1r = b1.reshape(1, hid)
    gammar = gamma.reshape(1, hid)
    betar = beta.reshape(1, hid)
    b2r = b2.reshape(1, dout)

    rep = lambda shape: pl.BlockSpec(shape, lambda i: (0, 0))
    return pl.pallas_call(
        _fused_mlp_block,
        grid=grid,
        in_specs=[
            pl.BlockSpec((IN_ROWS, din), lambda i: (i // SUBS, 0),
                         pipeline_mode=pl.Buffered(buffer_count=2)),
            rep((din, hid)),
            rep((1, hid)),
            rep((1, hid)),
            rep((1, hid)),
            rep((hid, dout)),
            rep((1, dout)),
        ],
        out_specs=pl.BlockSpec((OUT_ROWS, dout), lambda i: (i, 0)),
        out_shape=jax.ShapeDtypeStruct((n, dout), jnp.float32),
        compiler_params=pltpu.CompilerParams(
            dimension_semantics=("arbitrary",),
            vmem_limit_bytes=127 * 1024 * 1024,
        ),
    )(x, W1, b1r, gammar, betar, W2, b2r)


# speedup vs baseline: 1.4087x; 1.4087x over previous
import jax
import jax.numpy as jnp
from jax.experimental import pallas as pl
from jax.experimental.pallas import tpu as pltpu

ROWS_PER_BLOCK = 25000


def _fused_mlp_block(x_ref, w1t_ref, b1_ref, gamma_ref, beta_ref, w2t_ref,
                     b2_ref, out_ref):
    h = jax.lax.dot_general(x_ref[...], w1t_ref[...],
                            (((1,), (1,)), ((), ())),
                            preferred_element_type=jnp.float32)
    h = h + b1_ref[...]
    h = jnp.where(h >= 0, h, 0.01 * h)
    mu = jnp.mean(h, axis=-1, keepdims=True)
    var = jnp.mean((h - mu) ** 2, axis=-1, keepdims=True)
    h = (h - mu) * jax.lax.rsqrt(var + 1e-5) * gamma_ref[...] + beta_ref[...]
    out = jax.lax.dot_general(h, w2t_ref[...],
                              (((1,), (1,)), ((), ())),
                              preferred_element_type=jnp.float32)
    out_ref[...] = out + b2_ref[...]


@jax.jit
def kernel(x, W1, b1, gamma, beta, W2, b2):
    n, din = x.shape
    hid = W1.shape[0]
    dout = W2.shape[0]
    blk = ROWS_PER_BLOCK
    grid = (n // blk,)

    b1r = b1.reshape(1, hid)
    gammar = gamma.reshape(1, hid)
    betar = beta.reshape(1, hid)
    b2r = b2.reshape(1, dout)

    rep = lambda shape: pl.BlockSpec(shape, lambda i: (0, 0))
    return pl.pallas_call(
        _fused_mlp_block,
        grid=grid,
        in_specs=[
            pl.BlockSpec((blk, din), lambda i: (i, 0)),
            rep((din, hid)),
            rep((1, hid)),
            rep((1, hid)),
            rep((1, hid)),
            rep((hid, dout)),
            rep((1, dout)),
        ],
        out_specs=pl.BlockSpec((blk, dout), lambda i: (i, 0)),
        out_shape=jax.ShapeDtypeStruct((n, dout), jnp.float32),
        compiler_params=pltpu.CompilerParams(
            dimension_semantics=("parallel",),
            vmem_limit_bytes=127 * 1024 * 1024,
        ),
    )(x, W1, b1r, gammar, betar, W2, b2r)


# speedup vs baseline: 1.4098x; 1.0008x over previous
import jax
import jax.numpy as jnp
from jax.experimental import pallas as pl
from jax.experimental.pallas import tpu as pltpu

ROWS_PER_BLOCK = 25000


def _fused_mlp_block(x_ref, w1t_ref, b1_ref, gamma_ref, beta_ref, w2t_ref,
                     b2_ref, out_ref):
    h = jax.lax.dot_general(x_ref[...], w1t_ref[...],
                            (((1,), (1,)), ((), ())),
                            preferred_element_type=jnp.float32)
    h = h + b1_ref[...]
    h = jnp.where(h >= 0, h, 0.01 * h)
    mu = jnp.mean(h, axis=-1, keepdims=True)
    var = jnp.mean((h - mu) ** 2, axis=-1, keepdims=True)
    h = (h - mu) * jax.lax.rsqrt(var + 1e-5) * gamma_ref[...] + beta_ref[...]
    out = jax.lax.dot_general(h, w2t_ref[...],
                              (((1,), (1,)), ((), ())),
                              preferred_element_type=jnp.float32)
    out_ref[...] = out + b2_ref[...]


@jax.jit
def kernel(x, W1, b1, gamma, beta, W2, b2):
    n, din = x.shape
    hid = W1.shape[0]
    dout = W2.shape[0]
    blk = ROWS_PER_BLOCK
    grid = (n // blk,)

    b1r = b1.reshape(1, hid)
    gammar = gamma.reshape(1, hid)
    betar = beta.reshape(1, hid)
    b2r = b2.reshape(1, dout)

    rep = lambda shape: pl.BlockSpec(shape, lambda i: (0, 0))
    return pl.pallas_call(
        _fused_mlp_block,
        grid=grid,
        in_specs=[
            pl.BlockSpec((blk, din), lambda i: (i, 0)),
            rep((din, hid)),
            rep((1, hid)),
            rep((1, hid)),
            rep((1, hid)),
            rep((hid, dout)),
            rep((1, dout)),
        ],
        out_specs=pl.BlockSpec((blk, dout), lambda i: (i, 0)),
        out_shape=jax.ShapeDtypeStruct((n, dout), jnp.float32),
        compiler_params=pltpu.CompilerParams(
            dimension_semantics=("arbitrary",),
            vmem_limit_bytes=127 * 1024 * 1024,
        ),
    )(x, W1, b1r, gammar, betar, W2, b2r)
